# baseline (speedup 1.0000x reference)
import functools

import jax
import jax.numpy as jnp
from jax import lax
from jax.experimental import pallas as pl
from jax.experimental.pallas import tpu as pltpu
from jax.experimental.pallas import tpu_sc as plsc

NUM_FACTORS = 64
BATCH = 16384
NC, NS = 2, 16
NW = NC * NS
CHUNK = 128
B_PER_W = BATCH // NW
N_CHUNKS = B_PER_W // CHUNK

_sc_mesh = plsc.VectorSubcoreMesh(
    core_axis_name="c", subcore_axis_name="s", num_cores=NC, num_subcores=NS)


@functools.partial(
    pl.kernel,
    out_type=(
        jax.ShapeDtypeStruct((BATCH, NUM_FACTORS), jnp.float32),
        jax.ShapeDtypeStruct((BATCH, NUM_FACTORS), jnp.float32),
    ),
    mesh=_sc_mesh,
    scratch_types=[
        pltpu.VMEM((N_CHUNKS, CHUNK), jnp.int32),
        pltpu.VMEM((N_CHUNKS, CHUNK), jnp.int32),
        pltpu.VMEM((B_PER_W, NUM_FACTORS), jnp.float32),
        pltpu.VMEM((B_PER_W, NUM_FACTORS), jnp.float32),
        pltpu.SemaphoreType.DMA,
    ],
    compiler_params=pltpu.CompilerParams(use_tc_tiling_on_sc=False),
)
def _sc_gather(user_idx_hbm, item_idx_hbm, user_table_hbm, item_table_hbm,
               u_out_hbm, i_out_hbm, uidx_v, iidx_v, urows_v, irows_v, sem):
    wid = lax.axis_index("s") * NC + lax.axis_index("c")
    pltpu.sync_copy(user_idx_hbm.at[pl.ds(wid * N_CHUNKS, N_CHUNKS)], uidx_v)
    pltpu.sync_copy(item_idx_hbm.at[pl.ds(wid * N_CHUNKS, N_CHUNKS)], iidx_v)
    copies = []
    for j in range(N_CHUNKS):
        copies.append(pltpu.async_copy(
            user_table_hbm.at[uidx_v.at[j]],
            urows_v.at[pl.ds(j * CHUNK, CHUNK)], sem))
        copies.append(pltpu.async_copy(
            item_table_hbm.at[iidx_v.at[j]],
            irows_v.at[pl.ds(j * CHUNK, CHUNK)], sem))
    for c in copies:
        c.wait()
    base = wid * B_PER_W
    pltpu.sync_copy(urows_v, u_out_hbm.at[pl.ds(base, B_PER_W)])
    pltpu.sync_copy(irows_v, i_out_hbm.at[pl.ds(base, B_PER_W)])


_BLK = 2048


def _mlp_body(u_ref, i_ref, w1a_ref, w1b_ref, b1_ref, w2_ref, b2_ref,
              w3_ref, b3_ref, out_ref):
    h = jnp.dot(u_ref[...], w1a_ref[...], preferred_element_type=jnp.float32)
    h += jnp.dot(i_ref[...], w1b_ref[...], preferred_element_type=jnp.float32)
    h = jnp.maximum(h + b1_ref[...], 0.0)
    h2 = jnp.dot(h, w2_ref[...], preferred_element_type=jnp.float32)
    h2 = jnp.maximum(h2 + b2_ref[...], 0.0)
    out_ref[...] = jnp.dot(h2, w3_ref[...],
                           preferred_element_type=jnp.float32) + b3_ref[...]


def _mlp(u_emb, i_emb, W1a, W1b, b1, W2, b2, W3, b3):
    grid = (BATCH // _BLK,)
    full = lambda shape: pl.BlockSpec(shape, lambda i: (0, 0))
    return pl.pallas_call(
        _mlp_body,
        grid=grid,
        in_specs=[
            pl.BlockSpec((_BLK, NUM_FACTORS), lambda i: (i, 0)),
            pl.BlockSpec((_BLK, NUM_FACTORS), lambda i: (i, 0)),
            full(W1a.shape), full(W1b.shape), full(b1.shape),
            full(W2.shape), full(b2.shape), full(W3.shape), full(b3.shape),
        ],
        out_specs=pl.BlockSpec((_BLK, 1), lambda i: (i, 0)),
        out_shape=jax.ShapeDtypeStruct((BATCH, 1), jnp.float32),
    )(u_emb, i_emb, W1a, W1b, b1, W2, b2, W3, b3)


def kernel(user, item, user_table, item_table, W1, b1, W2, b2, W3, b3):
    user2d = user.astype(jnp.int32).reshape(BATCH // CHUNK, CHUNK)
    item2d = item.astype(jnp.int32).reshape(BATCH // CHUNK, CHUNK)
    u_emb, i_emb = _sc_gather(user2d, item2d, user_table, item_table)
    W1a = W1[:NUM_FACTORS]
    W1b = W1[NUM_FACTORS:]
    return _mlp(u_emb, i_emb, W1a, W1b, b1.reshape(1, -1), W2,
                b2.reshape(1, -1), W3, b3.reshape(1, -1))


# speedup vs baseline: 1.9483x; 1.9483x over previous
import functools

import jax
import jax.numpy as jnp
from jax import lax
from jax.experimental import pallas as pl
from jax.experimental.pallas import tpu as pltpu
from jax.experimental.pallas import tpu_sc as plsc

NUM_FACTORS = 64
BATCH = 16384
NUM_ROWS = 1000000
BLK_FOLD = 2048
NQ = 4
QBLOCKS = 123
OFFQ = QBLOCKS * BLK_FOLD
NCOLB = (NUM_ROWS + BLK_FOLD - 1) // BLK_FOLD

NC, NS = 2, 16
NW = NC * NS
CHUNK = 128
B_PER_W = BATCH // NW
N_CHUNKS = B_PER_W // CHUNK


def _fold_body(t_ref0, t_ref1, t_ref2, t_ref3, w_ref, out_ref):
    dn = (((0,), (0,)), ((), ()))
    parts = []
    for t_ref in (t_ref0, t_ref1, t_ref2, t_ref3):
        h = lax.dot_general(t_ref[...], w_ref[...], dn,
                            preferred_element_type=jnp.float32)
        parts.append(h)
    out_ref[...] = jnp.concatenate(parts, axis=1)


def _fold(tT, w_half):
    def col_map(q):
        return lambda i: (0, jnp.minimum(q * QBLOCKS + i, NCOLB - 1))

    return pl.pallas_call(
        _fold_body,
        grid=(QBLOCKS,),
        in_specs=[
            pl.BlockSpec((NUM_FACTORS, BLK_FOLD), col_map(0)),
            pl.BlockSpec((NUM_FACTORS, BLK_FOLD), col_map(1)),
            pl.BlockSpec((NUM_FACTORS, BLK_FOLD), col_map(2)),
            pl.BlockSpec((NUM_FACTORS, BLK_FOLD), col_map(3)),
            pl.BlockSpec((NUM_FACTORS, 32), lambda i: (0, 0)),
        ],
        out_specs=pl.BlockSpec((BLK_FOLD, 128), lambda i: (i, 0)),
        out_shape=jax.ShapeDtypeStruct((OFFQ, 128), jnp.float32),
        compiler_params=pltpu.CompilerParams(
            fuse_transposed_lhs_in_matmul=True),
    )(tT, tT, tT, tT, w_half)


_sc_mesh = plsc.VectorSubcoreMesh(
    core_axis_name="c", subcore_axis_name="s", num_cores=NC, num_subcores=NS)


@functools.partial(
    pl.kernel,
    out_type=(
        jax.ShapeDtypeStruct((BATCH, 128), jnp.float32),
        jax.ShapeDtypeStruct((BATCH, 128), jnp.float32),
    ),
    mesh=_sc_mesh,
    scratch_types=[
        pltpu.VMEM((N_CHUNKS, CHUNK), jnp.int32),
        pltpu.VMEM((N_CHUNKS, CHUNK), jnp.int32),
        pltpu.VMEM((B_PER_W, 128), jnp.float32),
        pltpu.SemaphoreType.DMA,
    ],
)
def _sc_gather(uidx_hbm, iidx_hbm, hu_hbm, hi_hbm,
               u_out_hbm, i_out_hbm, uidx_v, iidx_v, rows_v, sem):
    wid = lax.axis_index("s") * NC + lax.axis_index("c")
    base = wid * B_PER_W
    pltpu.sync_copy(uidx_hbm.at[pl.ds(wid * N_CHUNKS, N_CHUNKS)], uidx_v)
    pltpu.sync_copy(iidx_hbm.at[pl.ds(wid * N_CHUNKS, N_CHUNKS)], iidx_v)
    for idx_v, h_hbm, out_hbm in ((uidx_v, hu_hbm, u_out_hbm),
                                  (iidx_v, hi_hbm, i_out_hbm)):
        copies = []
        for j in range(N_CHUNKS):
            copies.append(pltpu.async_copy(
                h_hbm.at[idx_v.at[j]],
                rows_v.at[pl.ds(j * CHUNK, CHUNK)], sem))
        for c in copies:
            c.wait()
        pltpu.sync_copy(rows_v, out_hbm.at[pl.ds(base, B_PER_W)])


_BLK_MLP = 2048


def _select_quadrant(x128, q):
    p0 = x128[:, 0:32]
    p1 = x128[:, 32:64]
    p2 = x128[:, 64:96]
    p3 = x128[:, 96:128]
    lo = jnp.where(q == 0, p0, p1)
    hi = jnp.where(q == 2, p2, p3)
    return jnp.where(q < 2, lo, hi)


def _mlp_body(gu_ref, gi_ref, qu_ref, qi_ref, b1_ref, w2_ref, b2_ref,
              w3_ref, b3_ref, out_ref):
    hu = _select_quadrant(gu_ref[...], qu_ref[...])
    hi = _select_quadrant(gi_ref[...], qi_ref[...])
    h = jnp.maximum(hu + hi + b1_ref[...], 0.0)
    h2 = jnp.dot(h, w2_ref[...], preferred_element_type=jnp.float32)
    h2 = jnp.maximum(h2 + b2_ref[...], 0.0)
    out_ref[...] = jnp.dot(h2, w3_ref[...],
                           preferred_element_type=jnp.float32) + b3_ref[...]


def _mlp(gu, gi, qu, qi, b1r, W2, b2r, W3, b3r):
    full = lambda shape: pl.BlockSpec(shape, lambda i: (0, 0))
    return pl.pallas_call(
        _mlp_body,
        grid=(BATCH // _BLK_MLP,),
        in_specs=[
            pl.BlockSpec((_BLK_MLP, 128), lambda i: (i, 0)),
            pl.BlockSpec((_BLK_MLP, 128), lambda i: (i, 0)),
            pl.BlockSpec((_BLK_MLP, 1), lambda i: (i, 0)),
            pl.BlockSpec((_BLK_MLP, 1), lambda i: (i, 0)),
            full(b1r.shape), full(W2.shape), full(b2r.shape),
            full(W3.shape), full(b3r.shape),
        ],
        out_specs=pl.BlockSpec((_BLK_MLP, 1), lambda i: (i, 0)),
        out_shape=jax.ShapeDtypeStruct((BATCH, 1), jnp.float32),
    )(gu, gi, qu, qi, b1r, W2, b2r, W3, b3r)


def kernel(user, item, user_table, item_table, W1, b1, W2, b2, W3, b3):
    user = user.astype(jnp.int32)
    item = item.astype(jnp.int32)
    hu = _fold(user_table.T, W1[:NUM_FACTORS])
    hi = _fold(item_table.T, W1[NUM_FACTORS:])
    qu = user // OFFQ
    qi = item // OFFQ
    pu = (user - qu * OFFQ).reshape(BATCH // CHUNK, CHUNK)
    pi = (item - qi * OFFQ).reshape(BATCH // CHUNK, CHUNK)
    gu, gi = _sc_gather(pu, pi, hu, hi)
    return _mlp(gu, gi, qu.reshape(BATCH, 1), qi.reshape(BATCH, 1),
                b1.reshape(1, -1), W2, b2.reshape(1, -1), W3,
                b3.reshape(1, -1))


# speedup vs baseline: 2.7298x; 1.4012x over previous
import functools

import jax
import jax.numpy as jnp
from jax import lax
from jax.experimental import pallas as pl
from jax.experimental.pallas import tpu as pltpu
from jax.experimental.pallas import tpu_sc as plsc

NUM_FACTORS = 64
BATCH = 16384
NUM_ROWS = 1000000
BLK_FOLD = 2048
NQ = 4
QBLOCKS = 123
OFFQ = QBLOCKS * BLK_FOLD
NCOLB = (NUM_ROWS + BLK_FOLD - 1) // BLK_FOLD

NC, NS = 2, 16
NW = NC * NS
CHUNK = 128
B_PER_W = BATCH // NW
N_CHUNKS = B_PER_W // CHUNK


def _fold_body(t_ref0, t_ref1, t_ref2, t_ref3, w4_ref, out_ref):
    dn = (((0,), (0,)), ((), ()))
    x4 = jnp.concatenate(
        [t_ref0[...], t_ref1[...], t_ref2[...], t_ref3[...]], axis=0)
    out_ref[...] = lax.dot_general(x4, w4_ref[...], dn,
                                   preferred_element_type=jnp.float32)


def _fold(tT, w4):
    def col_map(q):
        return lambda i: (0, jnp.minimum(q * QBLOCKS + i, NCOLB - 1))

    return pl.pallas_call(
        _fold_body,
        grid=(QBLOCKS,),
        in_specs=[
            pl.BlockSpec((NUM_FACTORS, BLK_FOLD), col_map(0)),
            pl.BlockSpec((NUM_FACTORS, BLK_FOLD), col_map(1)),
            pl.BlockSpec((NUM_FACTORS, BLK_FOLD), col_map(2)),
            pl.BlockSpec((NUM_FACTORS, BLK_FOLD), col_map(3)),
            pl.BlockSpec((4 * NUM_FACTORS, 128), lambda i: (0, 0)),
        ],
        out_specs=pl.BlockSpec((BLK_FOLD, 128), lambda i: (i, 0)),
        out_shape=jax.ShapeDtypeStruct((OFFQ, 128), jnp.float32),
        compiler_params=pltpu.CompilerParams(
            fuse_transposed_lhs_in_matmul=True),
    )(tT, tT, tT, tT, w4)


def _block_diag_w(w_half):
    z = jnp.zeros((4 * NUM_FACTORS, 128), jnp.float32)
    for q in range(4):
        z = z.at[q * NUM_FACTORS:(q + 1) * NUM_FACTORS,
                 q * 32:(q + 1) * 32].set(w_half)
    return z


_sc_mesh = plsc.VectorSubcoreMesh(
    core_axis_name="c", subcore_axis_name="s", num_cores=NC, num_subcores=NS)


@functools.partial(
    pl.kernel,
    out_type=(
        jax.ShapeDtypeStruct((BATCH, 128), jnp.float32),
        jax.ShapeDtypeStruct((BATCH, 128), jnp.float32),
    ),
    mesh=_sc_mesh,
    scratch_types=[
        pltpu.VMEM((N_CHUNKS, CHUNK), jnp.int32),
        pltpu.VMEM((N_CHUNKS, CHUNK), jnp.int32),
        pltpu.VMEM((B_PER_W, 128), jnp.float32),
        pltpu.SemaphoreType.DMA,
    ],
)
def _sc_gather(uidx_hbm, iidx_hbm, hu_hbm, hi_hbm,
               u_out_hbm, i_out_hbm, uidx_v, iidx_v, rows_v, sem):
    wid = lax.axis_index("s") * NC + lax.axis_index("c")
    base = wid * B_PER_W
    pltpu.sync_copy(uidx_hbm.at[pl.ds(wid * N_CHUNKS, N_CHUNKS)], uidx_v)
    pltpu.sync_copy(iidx_hbm.at[pl.ds(wid * N_CHUNKS, N_CHUNKS)], iidx_v)
    for idx_v, h_hbm, out_hbm in ((uidx_v, hu_hbm, u_out_hbm),
                                  (iidx_v, hi_hbm, i_out_hbm)):
        copies = []
        for j in range(N_CHUNKS):
            copies.append(pltpu.async_copy(
                h_hbm.at[idx_v.at[j]],
                rows_v.at[pl.ds(j * CHUNK, CHUNK)], sem))
        for c in copies:
            c.wait()
        pltpu.sync_copy(rows_v, out_hbm.at[pl.ds(base, B_PER_W)])


_BLK_MLP = 2048


def _mlp_body(gu_ref, gi_ref, mu_ref, mi_ref, p_ref, b1_ref, w2_ref, b2_ref,
              w3_ref, b3_ref, out_ref):
    xu = gu_ref[...] * mu_ref[...].astype(jnp.float32)
    xi = gi_ref[...] * mi_ref[...].astype(jnp.float32)
    hu = jnp.dot(xu, p_ref[...], preferred_element_type=jnp.float32)
    hi = jnp.dot(xi, p_ref[...], preferred_element_type=jnp.float32)
    h = jnp.maximum(hu + hi + b1_ref[...], 0.0)
    h2 = jnp.dot(h, w2_ref[...], preferred_element_type=jnp.float32)
    h2 = jnp.maximum(h2 + b2_ref[...], 0.0)
    out_ref[...] = jnp.dot(h2, w3_ref[...],
                           preferred_element_type=jnp.float32) + b3_ref[...]


def _mlp(gu, gi, mu, mi, ptile, b1r, W2, b2r, W3, b3r):
    full = lambda shape: pl.BlockSpec(shape, lambda i: (0, 0))
    return pl.pallas_call(
        _mlp_body,
        grid=(BATCH // _BLK_MLP,),
        in_specs=[
            pl.BlockSpec((_BLK_MLP, 128), lambda i: (i, 0)),
            pl.BlockSpec((_BLK_MLP, 128), lambda i: (i, 0)),
            pl.BlockSpec((_BLK_MLP, 128), lambda i: (i, 0)),
            pl.BlockSpec((_BLK_MLP, 128), lambda i: (i, 0)),
            full(ptile.shape), full(b1r.shape), full(W2.shape),
            full(b2r.shape), full(W3.shape), full(b3r.shape),
        ],
        out_specs=pl.BlockSpec((_BLK_MLP, 1), lambda i: (i, 0)),
        out_shape=jax.ShapeDtypeStruct((BATCH, 1), jnp.float32),
    )(gu, gi, mu, mi, ptile, b1r, W2, b2r, W3, b3r)


def kernel(user, item, user_table, item_table, W1, b1, W2, b2, W3, b3):
    user = user.astype(jnp.int32)
    item = item.astype(jnp.int32)
    hu = _fold(user_table.T, _block_diag_w(W1[:NUM_FACTORS]))
    hi = _fold(item_table.T, _block_diag_w(W1[NUM_FACTORS:]))
    qu = user // OFFQ
    qi = item // OFFQ
    pu = (user - qu * OFFQ).reshape(BATCH // CHUNK, CHUNK)
    pi = (item - qi * OFFQ).reshape(BATCH // CHUNK, CHUNK)
    gu, gi = _sc_gather(pu, pi, hu, hi)
    lane_q = jnp.arange(128, dtype=jnp.int32)[None, :] // 32
    mu = (lane_q == qu[:, None]).astype(jnp.bfloat16)
    mi = (lane_q == qi[:, None]).astype(jnp.bfloat16)
    ptile = jnp.tile(jnp.eye(32, dtype=jnp.float32), (4, 1))
    return _mlp(gu, gi, mu, mi, ptile, b1.reshape(1, -1), W2,
                b2.reshape(1, -1), W3, b3.reshape(1, -1))


# speedup vs baseline: 2.7359x; 1.0022x over previous
import functools

import jax
import jax.numpy as jnp
from jax import lax
from jax.experimental import pallas as pl
from jax.experimental.pallas import tpu as pltpu
from jax.experimental.pallas import tpu_sc as plsc

NUM_FACTORS = 64
BATCH = 16384
NUM_ROWS = 1000000
BLK_FOLD = 2048
NQ = 4
QBLOCKS = 123
OFFQ = QBLOCKS * BLK_FOLD
NCOLB = (NUM_ROWS + BLK_FOLD - 1) // BLK_FOLD

NC, NS = 2, 16
NW = NC * NS
CHUNK = 128
B_PER_W = BATCH // NW
N_CHUNKS = B_PER_W // CHUNK


def _fold_body(t_ref0, t_ref1, t_ref2, t_ref3, w4_ref, out_ref):
    dn = (((0,), (0,)), ((), ()))
    x4 = jnp.concatenate(
        [t_ref0[...], t_ref1[...], t_ref2[...], t_ref3[...]], axis=0)
    out_ref[...] = lax.dot_general(x4, w4_ref[...], dn,
                                   preferred_element_type=jnp.float32)


def _fold(tT, w4):
    def col_map(q):
        return lambda i: (0, jnp.minimum(q * QBLOCKS + i, NCOLB - 1))

    return pl.pallas_call(
        _fold_body,
        grid=(QBLOCKS,),
        in_specs=[
            pl.BlockSpec((NUM_FACTORS, BLK_FOLD), col_map(0)),
            pl.BlockSpec((NUM_FACTORS, BLK_FOLD), col_map(1)),
            pl.BlockSpec((NUM_FACTORS, BLK_FOLD), col_map(2)),
            pl.BlockSpec((NUM_FACTORS, BLK_FOLD), col_map(3)),
            pl.BlockSpec((4 * NUM_FACTORS, 128), lambda i: (0, 0)),
        ],
        out_specs=pl.BlockSpec((BLK_FOLD, 128), lambda i: (i, 0)),
        out_shape=jax.ShapeDtypeStruct((OFFQ, 128), jnp.float32),
        compiler_params=pltpu.CompilerParams(
            fuse_transposed_lhs_in_matmul=True),
    )(tT, tT, tT, tT, w4)


def _block_diag_w(w_half):
    z = jnp.zeros((4 * NUM_FACTORS, 128), jnp.float32)
    for q in range(4):
        z = z.at[q * NUM_FACTORS:(q + 1) * NUM_FACTORS,
                 q * 32:(q + 1) * 32].set(w_half)
    return z


_sc_mesh = plsc.VectorSubcoreMesh(
    core_axis_name="c", subcore_axis_name="s", num_cores=NC, num_subcores=NS)


@functools.partial(
    pl.kernel,
    out_type=(
        jax.ShapeDtypeStruct((BATCH, 128), jnp.float32),
        jax.ShapeDtypeStruct((BATCH, 128), jnp.float32),
    ),
    mesh=_sc_mesh,
    scratch_types=[
        pltpu.VMEM((N_CHUNKS, CHUNK), jnp.int32),
        pltpu.VMEM((N_CHUNKS, CHUNK), jnp.int32),
        pltpu.VMEM((B_PER_W, 128), jnp.float32),
        pltpu.SemaphoreType.DMA,
    ],
)
def _sc_gather(uidx_hbm, iidx_hbm, hu_hbm, hi_hbm,
               u_out_hbm, i_out_hbm, uidx_v, iidx_v, rows_v, sem):
    wid = lax.axis_index("s") * NC + lax.axis_index("c")
    base = wid * B_PER_W
    pltpu.sync_copy(uidx_hbm.at[pl.ds(wid * N_CHUNKS, N_CHUNKS)], uidx_v)
    pltpu.sync_copy(iidx_hbm.at[pl.ds(wid * N_CHUNKS, N_CHUNKS)], iidx_v)
    for idx_v, h_hbm, out_hbm in ((uidx_v, hu_hbm, u_out_hbm),
                                  (iidx_v, hi_hbm, i_out_hbm)):
        copies = []
        for j in range(N_CHUNKS):
            copies.append(pltpu.async_copy(
                h_hbm.at[idx_v.at[j]],
                rows_v.at[pl.ds(j * CHUNK, CHUNK)], sem))
        for c in copies:
            c.wait()
        pltpu.sync_copy(rows_v, out_hbm.at[pl.ds(base, B_PER_W)])


_BLK_MLP = 2048


def _mlp_body(gu_ref, gi_ref, mu_ref, mi_ref, p_ref, b1_ref, w2_ref, b2_ref,
              w3_ref, b3_ref, out_ref):
    xu = gu_ref[...] * mu_ref[...].astype(jnp.float32)
    xi = gi_ref[...] * mi_ref[...].astype(jnp.float32)
    x = xu + xi
    del p_ref
    h = (x[:, 0:32] + x[:, 32:64]) + (x[:, 64:96] + x[:, 96:128])
    hu = h
    hi = 0.0
    h = jnp.maximum(hu + hi + b1_ref[...], 0.0)
    h2 = jnp.dot(h, w2_ref[...], preferred_element_type=jnp.float32)
    h2 = jnp.maximum(h2 + b2_ref[...], 0.0)
    out_ref[...] = jnp.dot(h2, w3_ref[...],
                           preferred_element_type=jnp.float32) + b3_ref[...]


def _mlp(gu, gi, mu, mi, ptile, b1r, W2, b2r, W3, b3r):
    full = lambda shape: pl.BlockSpec(shape, lambda i: (0, 0))
    return pl.pallas_call(
        _mlp_body,
        grid=(BATCH // _BLK_MLP,),
        in_specs=[
            pl.BlockSpec((_BLK_MLP, 128), lambda i: (i, 0)),
            pl.BlockSpec((_BLK_MLP, 128), lambda i: (i, 0)),
            pl.BlockSpec((_BLK_MLP, 128), lambda i: (i, 0)),
            pl.BlockSpec((_BLK_MLP, 128), lambda i: (i, 0)),
            full(ptile.shape), full(b1r.shape), full(W2.shape),
            full(b2r.shape), full(W3.shape), full(b3r.shape),
        ],
        out_specs=pl.BlockSpec((_BLK_MLP, 1), lambda i: (i, 0)),
        out_shape=jax.ShapeDtypeStruct((BATCH, 1), jnp.float32),
    )(gu, gi, mu, mi, ptile, b1r, W2, b2r, W3, b3r)


def kernel(user, item, user_table, item_table, W1, b1, W2, b2, W3, b3):
    user = user.astype(jnp.int32)
    item = item.astype(jnp.int32)
    hu = _fold(user_table.T, _block_diag_w(W1[:NUM_FACTORS]))
    hi = _fold(item_table.T, _block_diag_w(W1[NUM_FACTORS:]))
    qu = user // OFFQ
    qi = item // OFFQ
    pu = (user - qu * OFFQ).reshape(BATCH // CHUNK, CHUNK)
    pi = (item - qi * OFFQ).reshape(BATCH // CHUNK, CHUNK)
    gu, gi = _sc_gather(pu, pi, hu, hi)
    lane_q = jnp.arange(128, dtype=jnp.int32)[None, :] // 32
    mu = (lane_q == qu[:, None]).astype(jnp.bfloat16)
    mi = (lane_q == qi[:, None]).astype(jnp.bfloat16)
    ptile = jnp.tile(jnp.eye(32, dtype=jnp.float32), (4, 1))
    return _mlp(gu, gi, mu, mi, ptile, b1.reshape(1, -1), W2,
                b2.reshape(1, -1), W3, b3.reshape(1, -1))


# speedup vs baseline: 3.4980x; 1.2786x over previous
import functools

import jax
import jax.numpy as jnp
from jax import lax
from jax.experimental import pallas as pl
from jax.experimental.pallas import tpu as pltpu
from jax.experimental.pallas import tpu_sc as plsc

NUM_FACTORS = 64
BATCH = 16384
NUM_ROWS = 1000000
BLK_FOLD = 8192
NQ = 4
QBLOCKS = 31
OFFQ = QBLOCKS * BLK_FOLD
NCOLB = (NUM_ROWS + BLK_FOLD - 1) // BLK_FOLD

NC, NS = 2, 16
NW = NC * NS
CHUNK = 128
B_PER_W = BATCH // NW
N_CHUNKS = B_PER_W // CHUNK


def _fold_body(t_ref0, t_ref1, t_ref2, t_ref3, w4_ref, out_ref):
    dn = (((0,), (0,)), ((), ()))
    x4 = jnp.concatenate(
        [t_ref0[...], t_ref1[...], t_ref2[...], t_ref3[...]], axis=0)
    out_ref[...] = lax.dot_general(x4, w4_ref[...], dn,
                                   preferred_element_type=jnp.float32)


def _fold(tT, w4):
    def col_map(q):
        return lambda i: (0, jnp.minimum(q * QBLOCKS + i, NCOLB - 1))

    return pl.pallas_call(
        _fold_body,
        grid=(QBLOCKS,),
        in_specs=[
            pl.BlockSpec((NUM_FACTORS, BLK_FOLD), col_map(0)),
            pl.BlockSpec((NUM_FACTORS, BLK_FOLD), col_map(1)),
            pl.BlockSpec((NUM_FACTORS, BLK_FOLD), col_map(2)),
            pl.BlockSpec((NUM_FACTORS, BLK_FOLD), col_map(3)),
            pl.BlockSpec((4 * NUM_FACTORS, 128), lambda i: (0, 0)),
        ],
        out_specs=pl.BlockSpec((BLK_FOLD, 128), lambda i: (i, 0)),
        out_shape=jax.ShapeDtypeStruct((OFFQ, 128), jnp.float32),
        compiler_params=pltpu.CompilerParams(
            fuse_transposed_lhs_in_matmul=True),
    )(tT, tT, tT, tT, w4)


def _block_diag_w(w_half):
    z = jnp.zeros((4 * NUM_FACTORS, 128), jnp.float32)
    for q in range(4):
        z = z.at[q * NUM_FACTORS:(q + 1) * NUM_FACTORS,
                 q * 32:(q + 1) * 32].set(w_half)
    return z


_sc_mesh = plsc.VectorSubcoreMesh(
    core_axis_name="c", subcore_axis_name="s", num_cores=NC, num_subcores=NS)


@functools.partial(
    pl.kernel,
    out_type=(
        jax.ShapeDtypeStruct((BATCH, 128), jnp.float32),
        jax.ShapeDtypeStruct((BATCH, 128), jnp.float32),
    ),
    mesh=_sc_mesh,
    scratch_types=[
        pltpu.VMEM((N_CHUNKS, CHUNK), jnp.int32),
        pltpu.VMEM((N_CHUNKS, CHUNK), jnp.int32),
        pltpu.VMEM((B_PER_W, 128), jnp.float32),
        pltpu.SemaphoreType.DMA,
    ],
)
def _sc_gather(uidx_hbm, iidx_hbm, hu_hbm, hi_hbm,
               u_out_hbm, i_out_hbm, uidx_v, iidx_v, rows_v, sem):
    wid = lax.axis_index("s") * NC + lax.axis_index("c")
    base = wid * B_PER_W
    pltpu.sync_copy(uidx_hbm.at[pl.ds(wid * N_CHUNKS, N_CHUNKS)], uidx_v)
    pltpu.sync_copy(iidx_hbm.at[pl.ds(wid * N_CHUNKS, N_CHUNKS)], iidx_v)
    for idx_v, h_hbm, out_hbm in ((uidx_v, hu_hbm, u_out_hbm),
                                  (iidx_v, hi_hbm, i_out_hbm)):
        copies = []
        for j in range(N_CHUNKS):
            copies.append(pltpu.async_copy(
                h_hbm.at[idx_v.at[j]],
                rows_v.at[pl.ds(j * CHUNK, CHUNK)], sem))
        for c in copies:
            c.wait()
        pltpu.sync_copy(rows_v, out_hbm.at[pl.ds(base, B_PER_W)])


_BLK_MLP = 2048


def _mlp_body(gu_ref, gi_ref, mu_ref, mi_ref, p_ref, b1_ref, w2_ref, b2_ref,
              w3_ref, b3_ref, out_ref):
    xu = gu_ref[...] * mu_ref[...].astype(jnp.float32)
    xi = gi_ref[...] * mi_ref[...].astype(jnp.float32)
    x = xu + xi
    del p_ref
    h = (x[:, 0:32] + x[:, 32:64]) + (x[:, 64:96] + x[:, 96:128])
    hu = h
    hi = 0.0
    h = jnp.maximum(hu + hi + b1_ref[...], 0.0)
    h2 = jnp.dot(h, w2_ref[...], preferred_element_type=jnp.float32)
    h2 = jnp.maximum(h2 + b2_ref[...], 0.0)
    out_ref[...] = jnp.dot(h2, w3_ref[...],
                           preferred_element_type=jnp.float32) + b3_ref[...]


def _mlp(gu, gi, mu, mi, ptile, b1r, W2, b2r, W3, b3r):
    full = lambda shape: pl.BlockSpec(shape, lambda i: (0, 0))
    return pl.pallas_call(
        _mlp_body,
        grid=(BATCH // _BLK_MLP,),
        in_specs=[
            pl.BlockSpec((_BLK_MLP, 128), lambda i: (i, 0)),
            pl.BlockSpec((_BLK_MLP, 128), lambda i: (i, 0)),
            pl.BlockSpec((_BLK_MLP, 128), lambda i: (i, 0)),
            pl.BlockSpec((_BLK_MLP, 128), lambda i: (i, 0)),
            full(ptile.shape), full(b1r.shape), full(W2.shape),
            full(b2r.shape), full(W3.shape), full(b3r.shape),
        ],
        out_specs=pl.BlockSpec((_BLK_MLP, 1), lambda i: (i, 0)),
        out_shape=jax.ShapeDtypeStruct((BATCH, 1), jnp.float32),
    )(gu, gi, mu, mi, ptile, b1r, W2, b2r, W3, b3r)


def kernel(user, item, user_table, item_table, W1, b1, W2, b2, W3, b3):
    user = user.astype(jnp.int32)
    item = item.astype(jnp.int32)
    hu = _fold(user_table.T, _block_diag_w(W1[:NUM_FACTORS]))
    hi = _fold(item_table.T, _block_diag_w(W1[NUM_FACTORS:]))
    qu = user // OFFQ
    qi = item // OFFQ
    pu = (user - qu * OFFQ).reshape(BATCH // CHUNK, CHUNK)
    pi = (item - qi * OFFQ).reshape(BATCH // CHUNK, CHUNK)
    gu, gi = _sc_gather(pu, pi, hu, hi)
    lane_q = jnp.arange(128, dtype=jnp.int32)[None, :] // 32
    mu = (lane_q == qu[:, None]).astype(jnp.bfloat16)
    mi = (lane_q == qi[:, None]).astype(jnp.bfloat16)
    ptile = jnp.tile(jnp.eye(32, dtype=jnp.float32), (4, 1))
    return _mlp(gu, gi, mu, mi, ptile, b1.reshape(1, -1), W2,
                b2.reshape(1, -1), W3, b3.reshape(1, -1))


# speedup vs baseline: 3.6083x; 1.0315x over previous
import functools

import jax
import jax.numpy as jnp
from jax import lax
from jax.experimental import pallas as pl
from jax.experimental.pallas import tpu as pltpu
from jax.experimental.pallas import tpu_sc as plsc

NUM_FACTORS = 64
BATCH = 16384
NUM_ROWS = 1000000
BLK_FOLD = 16384
NQ = 4
QBLOCKS = 16
OFFQ = QBLOCKS * BLK_FOLD
NCOLB = (NUM_ROWS + BLK_FOLD - 1) // BLK_FOLD

NC, NS = 2, 16
NW = NC * NS
CHUNK = 128
B_PER_W = BATCH // NW
N_CHUNKS = B_PER_W // CHUNK


def _fold_body(t_ref0, t_ref1, t_ref2, t_ref3, w4_ref, out_ref):
    dn = (((0,), (0,)), ((), ()))
    x4 = jnp.concatenate(
        [t_ref0[...], t_ref1[...], t_ref2[...], t_ref3[...]], axis=0)
    out_ref[...] = lax.dot_general(x4, w4_ref[...], dn,
                                   preferred_element_type=jnp.float32)


def _fold(tT, w4):
    def col_map(q):
        return lambda i: (0, jnp.minimum(q * QBLOCKS + i, NCOLB - 1))

    return pl.pallas_call(
        _fold_body,
        grid=(QBLOCKS,),
        in_specs=[
            pl.BlockSpec((NUM_FACTORS, BLK_FOLD), col_map(0)),
            pl.BlockSpec((NUM_FACTORS, BLK_FOLD), col_map(1)),
            pl.BlockSpec((NUM_FACTORS, BLK_FOLD), col_map(2)),
            pl.BlockSpec((NUM_FACTORS, BLK_FOLD), col_map(3)),
            pl.BlockSpec((4 * NUM_FACTORS, 128), lambda i: (0, 0)),
        ],
        out_specs=pl.BlockSpec((BLK_FOLD, 128), lambda i: (i, 0)),
        out_shape=jax.ShapeDtypeStruct((OFFQ, 128), jnp.float32),
        compiler_params=pltpu.CompilerParams(
            fuse_transposed_lhs_in_matmul=True),
    )(tT, tT, tT, tT, w4)


def _block_diag_w(w_half):
    z = jnp.zeros((4 * NUM_FACTORS, 128), jnp.float32)
    for q in range(4):
        z = z.at[q * NUM_FACTORS:(q + 1) * NUM_FACTORS,
                 q * 32:(q + 1) * 32].set(w_half)
    return z


_sc_mesh = plsc.VectorSubcoreMesh(
    core_axis_name="c", subcore_axis_name="s", num_cores=NC, num_subcores=NS)


@functools.partial(
    pl.kernel,
    out_type=(
        jax.ShapeDtypeStruct((BATCH, 128), jnp.float32),
        jax.ShapeDtypeStruct((BATCH, 128), jnp.float32),
    ),
    mesh=_sc_mesh,
    scratch_types=[
        pltpu.VMEM((N_CHUNKS, CHUNK), jnp.int32),
        pltpu.VMEM((N_CHUNKS, CHUNK), jnp.int32),
        pltpu.VMEM((B_PER_W, 128), jnp.float32),
        pltpu.SemaphoreType.DMA,
    ],
)
def _sc_gather(uidx_hbm, iidx_hbm, hu_hbm, hi_hbm,
               u_out_hbm, i_out_hbm, uidx_v, iidx_v, rows_v, sem):
    wid = lax.axis_index("s") * NC + lax.axis_index("c")
    base = wid * B_PER_W
    pltpu.sync_copy(uidx_hbm.at[pl.ds(wid * N_CHUNKS, N_CHUNKS)], uidx_v)
    pltpu.sync_copy(iidx_hbm.at[pl.ds(wid * N_CHUNKS, N_CHUNKS)], iidx_v)
    for idx_v, h_hbm, out_hbm in ((uidx_v, hu_hbm, u_out_hbm),
                                  (iidx_v, hi_hbm, i_out_hbm)):
        copies = []
        for j in range(N_CHUNKS):
            copies.append(pltpu.async_copy(
                h_hbm.at[idx_v.at[j]],
                rows_v.at[pl.ds(j * CHUNK, CHUNK)], sem))
        for c in copies:
            c.wait()
        pltpu.sync_copy(rows_v, out_hbm.at[pl.ds(base, B_PER_W)])


_BLK_MLP = 2048


def _mlp_body(gu_ref, gi_ref, mu_ref, mi_ref, p_ref, b1_ref, w2_ref, b2_ref,
              w3_ref, b3_ref, out_ref):
    xu = gu_ref[...] * mu_ref[...].astype(jnp.float32)
    xi = gi_ref[...] * mi_ref[...].astype(jnp.float32)
    x = xu + xi
    del p_ref
    h = (x[:, 0:32] + x[:, 32:64]) + (x[:, 64:96] + x[:, 96:128])
    hu = h
    hi = 0.0
    h = jnp.maximum(hu + hi + b1_ref[...], 0.0)
    h2 = jnp.dot(h, w2_ref[...], preferred_element_type=jnp.float32)
    h2 = jnp.maximum(h2 + b2_ref[...], 0.0)
    out_ref[...] = jnp.dot(h2, w3_ref[...],
                           preferred_element_type=jnp.float32) + b3_ref[...]


def _mlp(gu, gi, mu, mi, ptile, b1r, W2, b2r, W3, b3r):
    full = lambda shape: pl.BlockSpec(shape, lambda i: (0, 0))
    return pl.pallas_call(
        _mlp_body,
        grid=(BATCH // _BLK_MLP,),
        in_specs=[
            pl.BlockSpec((_BLK_MLP, 128), lambda i: (i, 0)),
            pl.BlockSpec((_BLK_MLP, 128), lambda i: (i, 0)),
            pl.BlockSpec((_BLK_MLP, 128), lambda i: (i, 0)),
            pl.BlockSpec((_BLK_MLP, 128), lambda i: (i, 0)),
            full(ptile.shape), full(b1r.shape), full(W2.shape),
            full(b2r.shape), full(W3.shape), full(b3r.shape),
        ],
        out_specs=pl.BlockSpec((_BLK_MLP, 1), lambda i: (i, 0)),
        out_shape=jax.ShapeDtypeStruct((BATCH, 1), jnp.float32),
    )(gu, gi, mu, mi, ptile, b1r, W2, b2r, W3, b3r)


def kernel(user, item, user_table, item_table, W1, b1, W2, b2, W3, b3):
    user = user.astype(jnp.int32)
    item = item.astype(jnp.int32)
    hu = _fold(user_table.T, _block_diag_w(W1[:NUM_FACTORS]))
    hi = _fold(item_table.T, _block_diag_w(W1[NUM_FACTORS:]))
    qu = user // OFFQ
    qi = item // OFFQ
    pu = (user - qu * OFFQ).reshape(BATCH // CHUNK, CHUNK)
    pi = (item - qi * OFFQ).reshape(BATCH // CHUNK, CHUNK)
    gu, gi = _sc_gather(pu, pi, hu, hi)
    lane_q = jnp.arange(128, dtype=jnp.int32)[None, :] // 32
    mu = (lane_q == qu[:, None]).astype(jnp.bfloat16)
    mi = (lane_q == qi[:, None]).astype(jnp.bfloat16)
    ptile = jnp.tile(jnp.eye(32, dtype=jnp.float32), (4, 1))
    return _mlp(gu, gi, mu, mi, ptile, b1.reshape(1, -1), W2,
                b2.reshape(1, -1), W3, b3.reshape(1, -1))


# speedup vs baseline: 3.9622x; 1.0981x over previous
import functools

import jax
import jax.numpy as jnp
from jax import lax
from jax.experimental import pallas as pl
from jax.experimental.pallas import tpu as pltpu
from jax.experimental.pallas import tpu_sc as plsc

NUM_FACTORS = 64
BATCH = 16384
NUM_ROWS = 1000000
BLK_FOLD = 4096
NQ = 8
QBLOCKS = 32
OFF8 = QBLOCKS * BLK_FOLD
NCOLB = (NUM_ROWS + BLK_FOLD - 1) // BLK_FOLD

NC, NS = 2, 16
NW = NC * NS
CHUNK = 128
B_PER_W = BATCH // NW
N_CHUNKS = B_PER_W // CHUNK


def _fold_body(t_ref0, t_ref1, t_ref2, t_ref3, t_ref4, t_ref5, t_ref6,
               t_ref7, w8_ref, out_ref):
    dn = (((0,), (0,)), ((), ()))
    x8 = jnp.concatenate(
        [t[...] for t in (t_ref0, t_ref1, t_ref2, t_ref3,
                          t_ref4, t_ref5, t_ref6, t_ref7)],
        axis=0).astype(jnp.bfloat16)
    h = lax.dot_general(x8, w8_ref[...], dn,
                        preferred_element_type=jnp.float32)
    a_bits = lax.bitcast_convert_type(h[:, :128], jnp.uint32)
    b_bits = lax.bitcast_convert_type(h[:, 128:], jnp.uint32)
    half = jnp.uint32(0x8000)
    lo = (a_bits + half) >> 16
    hi = (b_bits + half) & jnp.uint32(0xFFFF0000)
    out_ref[...] = lax.bitcast_convert_type(hi | lo, jnp.int32)


def _fold(tT, w8):
    def col_map(q):
        return lambda i: (0, jnp.minimum(q * QBLOCKS + i, NCOLB - 1))

    return pl.pallas_call(
        _fold_body,
        grid=(QBLOCKS,),
        in_specs=[pl.BlockSpec((NUM_FACTORS, BLK_FOLD), col_map(q))
                  for q in range(NQ)] +
                 [pl.BlockSpec((NQ * NUM_FACTORS, 256), lambda i: (0, 0))],
        out_specs=pl.BlockSpec((BLK_FOLD, 128), lambda i: (i, 0)),
        out_shape=jax.ShapeDtypeStruct((OFF8, 128), jnp.int32),
        compiler_params=pltpu.CompilerParams(
            fuse_transposed_lhs_in_matmul=True),
    )(*([tT] * NQ), w8)


def _block_diag_w(w_half):
    z = jnp.zeros((NQ * NUM_FACTORS, 256), jnp.float32)
    for q in range(NQ):
        z = z.at[q * NUM_FACTORS:(q + 1) * NUM_FACTORS,
                 q * 32:(q + 1) * 32].set(w_half)
    return z.astype(jnp.bfloat16)


_sc_mesh = plsc.VectorSubcoreMesh(
    core_axis_name="c", subcore_axis_name="s", num_cores=NC, num_subcores=NS)


@functools.partial(
    pl.kernel,
    out_type=(
        jax.ShapeDtypeStruct((BATCH, 128), jnp.int32),
        jax.ShapeDtypeStruct((BATCH, 128), jnp.int32),
    ),
    mesh=_sc_mesh,
    scratch_types=[
        pltpu.VMEM((N_CHUNKS, CHUNK), jnp.int32),
        pltpu.VMEM((N_CHUNKS, CHUNK), jnp.int32),
        pltpu.VMEM((B_PER_W, 128), jnp.int32),
        pltpu.SemaphoreType.DMA,
    ],
)
def _sc_gather(uidx_hbm, iidx_hbm, hu_hbm, hi_hbm,
               u_out_hbm, i_out_hbm, uidx_v, iidx_v, rows_v, sem):
    wid = lax.axis_index("s") * NC + lax.axis_index("c")
    base = wid * B_PER_W
    pltpu.sync_copy(uidx_hbm.at[pl.ds(wid * N_CHUNKS, N_CHUNKS)], uidx_v)
    pltpu.sync_copy(iidx_hbm.at[pl.ds(wid * N_CHUNKS, N_CHUNKS)], iidx_v)
    for idx_v, h_hbm, out_hbm in ((uidx_v, hu_hbm, u_out_hbm),
                                  (iidx_v, hi_hbm, i_out_hbm)):
        copies = []
        for j in range(N_CHUNKS):
            copies.append(pltpu.async_copy(
                h_hbm.at[idx_v.at[j]],
                rows_v.at[pl.ds(j * CHUNK, CHUNK)], sem))
        for c in copies:
            c.wait()
        pltpu.sync_copy(rows_v, out_hbm.at[pl.ds(base, B_PER_W)])


_BLK_MLP = 2048


def _unpack_select(g_ref, m_ref):
    g = g_ref[...]
    a = lax.bitcast_convert_type(g << 16, jnp.float32)
    b = lax.bitcast_convert_type(
        lax.bitcast_convert_type(g, jnp.uint32) & jnp.uint32(0xFFFF0000),
        jnp.float32)
    m = m_ref[...]
    x = a * (m == 1).astype(jnp.float32) + b * (m == 2).astype(jnp.float32)
    return (x[:, 0:32] + x[:, 32:64]) + (x[:, 64:96] + x[:, 96:128])


def _mlp_body(gu_ref, gi_ref, mu_ref, mi_ref, b1_ref, w2_ref, b2_ref,
              w3_ref, b3_ref, out_ref):
    h = _unpack_select(gu_ref, mu_ref) + _unpack_select(gi_ref, mi_ref)
    h = jnp.maximum(h + b1_ref[...], 0.0)
    h2 = jnp.dot(h, w2_ref[...], preferred_element_type=jnp.float32)
    h2 = jnp.maximum(h2 + b2_ref[...], 0.0)
    out_ref[...] = jnp.dot(h2, w3_ref[...],
                           preferred_element_type=jnp.float32) + b3_ref[...]


def _mlp(gu, gi, mu, mi, b1r, W2, b2r, W3, b3r):
    full = lambda shape: pl.BlockSpec(shape, lambda i: (0, 0))
    return pl.pallas_call(
        _mlp_body,
        grid=(BATCH // _BLK_MLP,),
        in_specs=[
            pl.BlockSpec((_BLK_MLP, 128), lambda i: (i, 0)),
            pl.BlockSpec((_BLK_MLP, 128), lambda i: (i, 0)),
            pl.BlockSpec((_BLK_MLP, 128), lambda i: (i, 0)),
            pl.BlockSpec((_BLK_MLP, 128), lambda i: (i, 0)),
            full(b1r.shape), full(W2.shape), full(b2r.shape),
            full(W3.shape), full(b3r.shape),
        ],
        out_specs=pl.BlockSpec((_BLK_MLP, 1), lambda i: (i, 0)),
        out_shape=jax.ShapeDtypeStruct((BATCH, 1), jnp.float32),
    )(gu, gi, mu, mi, b1r, W2, b2r, W3, b3r)


def _lane_mask(q8):
    lane_q = jnp.arange(128, dtype=jnp.int32)[None, :] // 32
    sel = lane_q == (q8 & 3)[:, None]
    stream = 1 + (q8 >> 2)[:, None]
    return jnp.where(sel, stream, 0).astype(jnp.int8)


def kernel(user, item, user_table, item_table, W1, b1, W2, b2, W3, b3):
    user = user.astype(jnp.int32)
    item = item.astype(jnp.int32)
    hu = _fold(user_table.T, _block_diag_w(W1[:NUM_FACTORS]))
    hi = _fold(item_table.T, _block_diag_w(W1[NUM_FACTORS:]))
    q8u = user >> 17
    q8i = item >> 17
    pu = (user & (OFF8 - 1)).reshape(BATCH // CHUNK, CHUNK)
    pi = (item & (OFF8 - 1)).reshape(BATCH // CHUNK, CHUNK)
    gu, gi = _sc_gather(pu, pi, hu, hi)
    return _mlp(gu, gi, _lane_mask(q8u), _lane_mask(q8i),
                b1.reshape(1, -1), W2, b2.reshape(1, -1), W3,
                b3.reshape(1, -1))


# speedup vs baseline: 4.0845x; 1.0309x over previous
import functools

import jax
import jax.numpy as jnp
from jax import lax
from jax.experimental import pallas as pl
from jax.experimental.pallas import tpu as pltpu
from jax.experimental.pallas import tpu_sc as plsc

NUM_FACTORS = 64
BATCH = 16384
NUM_ROWS = 1000000
BLK_FOLD = 8192
NQ = 8
QBLOCKS = 16
OFF8 = QBLOCKS * BLK_FOLD
NCOLB = (NUM_ROWS + BLK_FOLD - 1) // BLK_FOLD

NC, NS = 2, 16
NW = NC * NS
CHUNK = 128
B_PER_W = BATCH // NW
N_CHUNKS = B_PER_W // CHUNK


def _fold_body(t_ref0, t_ref1, t_ref2, t_ref3, t_ref4, t_ref5, t_ref6,
               t_ref7, w8_ref, out_ref):
    dn = (((0,), (0,)), ((), ()))
    x8 = jnp.concatenate(
        [t[...] for t in (t_ref0, t_ref1, t_ref2, t_ref3,
                          t_ref4, t_ref5, t_ref6, t_ref7)],
        axis=0).astype(jnp.bfloat16)
    h = lax.dot_general(x8, w8_ref[...], dn,
                        preferred_element_type=jnp.float32)
    a_bits = lax.bitcast_convert_type(h[:, :128], jnp.uint32)
    b_bits = lax.bitcast_convert_type(h[:, 128:], jnp.uint32)
    half = jnp.uint32(0x8000)
    lo = (a_bits + half) >> 16
    hi = (b_bits + half) & jnp.uint32(0xFFFF0000)
    out_ref[...] = lax.bitcast_convert_type(hi | lo, jnp.int32)


def _fold(tT, w8):
    def col_map(q):
        return lambda i: (0, jnp.minimum(q * QBLOCKS + i, NCOLB - 1))

    return pl.pallas_call(
        _fold_body,
        grid=(QBLOCKS,),
        in_specs=[pl.BlockSpec((NUM_FACTORS, BLK_FOLD), col_map(q))
                  for q in range(NQ)] +
                 [pl.BlockSpec((NQ * NUM_FACTORS, 256), lambda i: (0, 0))],
        out_specs=pl.BlockSpec((BLK_FOLD, 128), lambda i: (i, 0)),
        out_shape=jax.ShapeDtypeStruct((OFF8, 128), jnp.int32),
        compiler_params=pltpu.CompilerParams(
            fuse_transposed_lhs_in_matmul=True),
    )(*([tT] * NQ), w8)


def _block_diag_w(w_half):
    z = jnp.zeros((NQ * NUM_FACTORS, 256), jnp.float32)
    for q in range(NQ):
        z = z.at[q * NUM_FACTORS:(q + 1) * NUM_FACTORS,
                 q * 32:(q + 1) * 32].set(w_half)
    return z.astype(jnp.bfloat16)


_sc_mesh = plsc.VectorSubcoreMesh(
    core_axis_name="c", subcore_axis_name="s", num_cores=NC, num_subcores=NS)


@functools.partial(
    pl.kernel,
    out_type=(
        jax.ShapeDtypeStruct((BATCH, 128), jnp.int32),
        jax.ShapeDtypeStruct((BATCH, 128), jnp.int32),
    ),
    mesh=_sc_mesh,
    scratch_types=[
        pltpu.VMEM((N_CHUNKS, CHUNK), jnp.int32),
        pltpu.VMEM((N_CHUNKS, CHUNK), jnp.int32),
        pltpu.VMEM((B_PER_W, 128), jnp.int32),
        pltpu.SemaphoreType.DMA,
    ],
)
def _sc_gather(uidx_hbm, iidx_hbm, hu_hbm, hi_hbm,
               u_out_hbm, i_out_hbm, uidx_v, iidx_v, rows_v, sem):
    wid = lax.axis_index("s") * NC + lax.axis_index("c")
    base = wid * B_PER_W
    pltpu.sync_copy(uidx_hbm.at[pl.ds(wid * N_CHUNKS, N_CHUNKS)], uidx_v)
    pltpu.sync_copy(iidx_hbm.at[pl.ds(wid * N_CHUNKS, N_CHUNKS)], iidx_v)
    for idx_v, h_hbm, out_hbm in ((uidx_v, hu_hbm, u_out_hbm),
                                  (iidx_v, hi_hbm, i_out_hbm)):
        copies = []
        for j in range(N_CHUNKS):
            copies.append(pltpu.async_copy(
                h_hbm.at[idx_v.at[j]],
                rows_v.at[pl.ds(j * CHUNK, CHUNK)], sem))
        for c in copies:
            c.wait()
        pltpu.sync_copy(rows_v, out_hbm.at[pl.ds(base, B_PER_W)])


_BLK_MLP = 4096


def _unpack_select(g_ref, m_ref):
    g = g_ref[...]
    a = lax.bitcast_convert_type(g << 16, jnp.float32)
    b = lax.bitcast_convert_type(
        lax.bitcast_convert_type(g, jnp.uint32) & jnp.uint32(0xFFFF0000),
        jnp.float32)
    m = m_ref[...]
    x = a * (m == 1).astype(jnp.float32) + b * (m == 2).astype(jnp.float32)
    return (x[:, 0:32] + x[:, 32:64]) + (x[:, 64:96] + x[:, 96:128])


def _mlp_body(gu_ref, gi_ref, mu_ref, mi_ref, b1_ref, w2_ref, b2_ref,
              w3_ref, b3_ref, out_ref):
    h = _unpack_select(gu_ref, mu_ref) + _unpack_select(gi_ref, mi_ref)
    h = jnp.maximum(h + b1_ref[...], 0.0)
    h2 = jnp.dot(h, w2_ref[...], preferred_element_type=jnp.float32)
    h2 = jnp.maximum(h2 + b2_ref[...], 0.0)
    out_ref[...] = jnp.dot(h2, w3_ref[...],
                           preferred_element_type=jnp.float32) + b3_ref[...]


def _mlp(gu, gi, mu, mi, b1r, W2, b2r, W3, b3r):
    full = lambda shape: pl.BlockSpec(shape, lambda i: (0, 0))
    return pl.pallas_call(
        _mlp_body,
        grid=(BATCH // _BLK_MLP,),
        in_specs=[
            pl.BlockSpec((_BLK_MLP, 128), lambda i: (i, 0)),
            pl.BlockSpec((_BLK_MLP, 128), lambda i: (i, 0)),
            pl.BlockSpec((_BLK_MLP, 128), lambda i: (i, 0)),
            pl.BlockSpec((_BLK_MLP, 128), lambda i: (i, 0)),
            full(b1r.shape), full(W2.shape), full(b2r.shape),
            full(W3.shape), full(b3r.shape),
        ],
        out_specs=pl.BlockSpec((_BLK_MLP, 1), lambda i: (i, 0)),
        out_shape=jax.ShapeDtypeStruct((BATCH, 1), jnp.float32),
    )(gu, gi, mu, mi, b1r, W2, b2r, W3, b3r)


def _lane_mask(q8):
    lane_q = jnp.arange(128, dtype=jnp.int32)[None, :] // 32
    sel = lane_q == (q8 & 3)[:, None]
    stream = 1 + (q8 >> 2)[:, None]
    return jnp.where(sel, stream, 0).astype(jnp.int8)


def kernel(user, item, user_table, item_table, W1, b1, W2, b2, W3, b3):
    user = user.astype(jnp.int32)
    item = item.astype(jnp.int32)
    hu = _fold(user_table.T, _block_diag_w(W1[:NUM_FACTORS]))
    hi = _fold(item_table.T, _block_diag_w(W1[NUM_FACTORS:]))
    q8u = user >> 17
    q8i = item >> 17
    pu = (user & (OFF8 - 1)).reshape(BATCH // CHUNK, CHUNK)
    pi = (item & (OFF8 - 1)).reshape(BATCH // CHUNK, CHUNK)
    gu, gi = _sc_gather(pu, pi, hu, hi)
    return _mlp(gu, gi, _lane_mask(q8u), _lane_mask(q8i),
                b1.reshape(1, -1), W2, b2.reshape(1, -1), W3,
                b3.reshape(1, -1))
